# Initial kernel scaffold; baseline (speedup 1.0000x reference)
import jax
import jax.numpy as jnp
from jax.experimental import pallas as pl


def kernel(x, w, b):
    raise NotImplementedError("write your pallas kernel here")


# speedup vs baseline: 2.7680x; 2.7680x over previous
import functools

import jax
import jax.numpy as jnp
from jax.experimental import pallas as pl
from jax.experimental.pallas import tpu as pltpu

_BM = 1024


def _linear_kernel(x_ref, wt_ref, b_ref, o_ref):
    xb = x_ref[...].astype(jnp.bfloat16)
    o_ref[...] = (
        jnp.dot(xb, wt_ref[...], preferred_element_type=jnp.float32)
        + b_ref[...]
    )


@functools.partial(jax.jit, static_argnames=("bm",))
def _forward(x, w, b, *, bm):
    M, K = x.shape
    N = w.shape[0]
    wt = w.T.astype(jnp.bfloat16)
    b_row = b.reshape(1, N)
    grid = (M // bm,)
    out = pl.pallas_call(
        _linear_kernel,
        out_shape=jax.ShapeDtypeStruct((M, N), jnp.float32),
        grid=grid,
        in_specs=[
            pl.BlockSpec((bm, K), lambda i: (i, 0)),
            pl.BlockSpec((K, N), lambda i: (0, 0)),
            pl.BlockSpec((1, N), lambda i: (0, 0)),
        ],
        out_specs=pl.BlockSpec((bm, N), lambda i: (i, 0)),
        compiler_params=pltpu.CompilerParams(
            dimension_semantics=("parallel",)),
        cost_estimate=pl.CostEstimate(
            flops=2 * M * N * K,
            bytes_accessed=4 * M * K + 2 * K * N + 4 * M * N,
            transcendentals=0),
    )(x, wt, b_row)
    return out


def kernel(x, w, b):
    bm = _BM if x.shape[0] % _BM == 0 else 8
    return _forward(x, w, b, bm=bm)


# speedup vs baseline: 2.7885x; 1.0074x over previous
import functools

import jax
import jax.numpy as jnp
from jax.experimental import pallas as pl
from jax.experimental.pallas import tpu as pltpu

_BM = 512


def _linear_kernel(x_ref, wt_ref, b_ref, o_ref):
    xb = x_ref[...].astype(jnp.bfloat16)
    o_ref[...] = (
        jnp.dot(xb, wt_ref[...], preferred_element_type=jnp.float32)
        + b_ref[...]
    )


@functools.partial(jax.jit, static_argnames=("bm",))
def _forward(x, w, b, *, bm):
    M, K = x.shape
    N = w.shape[0]
    wt = w.T.astype(jnp.bfloat16)
    b_row = b.reshape(1, N)
    grid = (M // bm,)
    out = pl.pallas_call(
        _linear_kernel,
        out_shape=jax.ShapeDtypeStruct((M, N), jnp.float32),
        grid=grid,
        in_specs=[
            pl.BlockSpec((bm, K), lambda i: (i, 0)),
            pl.BlockSpec((K, N), lambda i: (0, 0)),
            pl.BlockSpec((1, N), lambda i: (0, 0)),
        ],
        out_specs=pl.BlockSpec((bm, N), lambda i: (i, 0)),
        compiler_params=pltpu.CompilerParams(
            dimension_semantics=("parallel",)),
        cost_estimate=pl.CostEstimate(
            flops=2 * M * N * K,
            bytes_accessed=4 * M * K + 2 * K * N + 4 * M * N,
            transcendentals=0),
    )(x, wt, b_row)
    return out


def kernel(x, w, b):
    bm = _BM if x.shape[0] % _BM == 0 else 8
    return _forward(x, w, b, bm=bm)


# speedup vs baseline: 3.0521x; 1.0945x over previous
import functools

import jax
import jax.numpy as jnp
from jax.experimental import pallas as pl
from jax.experimental.pallas import tpu as pltpu

_BM = 512


def _fused_kernel(w_ref, x_ref, b_ref, o_ref, wb_ref):
    @pl.when(pl.program_id(1) == 0)
    def _():
        wb_ref[...] = w_ref[...].astype(jnp.bfloat16)

    xb = x_ref[...].astype(jnp.bfloat16)
    acc = jax.lax.dot_general(
        xb, wb_ref[...],
        dimension_numbers=(((1,), (1,)), ((), ())),
        preferred_element_type=jnp.float32)
    o_ref[...] = acc + b_ref[...]


@functools.partial(jax.jit, static_argnames=("bm",))
def _forward(x, w, b, *, bm):
    M, K = x.shape
    N = w.shape[0]
    b_row = b.reshape(1, N)
    steps = M // bm // 2
    grid = (2, steps)
    out = pl.pallas_call(
        _fused_kernel,
        out_shape=jax.ShapeDtypeStruct((M, N), jnp.float32),
        grid=grid,
        in_specs=[
            pl.BlockSpec((N, K), lambda i, j: (0, 0)),
            pl.BlockSpec((bm, K), lambda i, j: (i * steps + j, 0)),
            pl.BlockSpec((1, N), lambda i, j: (0, 0)),
        ],
        out_specs=pl.BlockSpec((bm, N), lambda i, j: (i * steps + j, 0)),
        scratch_shapes=[pltpu.VMEM((N, K), jnp.bfloat16)],
        compiler_params=pltpu.CompilerParams(
            dimension_semantics=("parallel", "arbitrary")),
        cost_estimate=pl.CostEstimate(
            flops=2 * M * N * K,
            bytes_accessed=4 * M * K + 4 * K * N + 4 * M * N,
            transcendentals=0),
    )(w, x, b_row)
    return out


def kernel(x, w, b):
    bm = _BM if x.shape[0] % (2 * _BM) == 0 else 8
    return _forward(x, w, b, bm=bm)


# speedup vs baseline: 3.0840x; 1.0104x over previous
import functools

import jax
import jax.numpy as jnp
from jax.experimental import pallas as pl
from jax.experimental.pallas import tpu as pltpu

_BM = 1024


def _fused_kernel(w_ref, x_ref, b_ref, o_ref, wb_ref):
    @pl.when(pl.program_id(1) == 0)
    def _():
        wb_ref[...] = w_ref[...].astype(jnp.bfloat16)

    xb = x_ref[...].astype(jnp.bfloat16)
    acc = jax.lax.dot_general(
        xb, wb_ref[...],
        dimension_numbers=(((1,), (1,)), ((), ())),
        preferred_element_type=jnp.float32)
    o_ref[...] = acc + b_ref[...]


@functools.partial(jax.jit, static_argnames=("bm",))
def _forward(x, w, b, *, bm):
    M, K = x.shape
    N = w.shape[0]
    b_row = b.reshape(1, N)
    steps = M // bm // 2
    grid = (2, steps)
    out = pl.pallas_call(
        _fused_kernel,
        out_shape=jax.ShapeDtypeStruct((M, N), jnp.float32),
        grid=grid,
        in_specs=[
            pl.BlockSpec((N, K), lambda i, j: (0, 0)),
            pl.BlockSpec((bm, K), lambda i, j: (i * steps + j, 0)),
            pl.BlockSpec((1, N), lambda i, j: (0, 0)),
        ],
        out_specs=pl.BlockSpec((bm, N), lambda i, j: (i * steps + j, 0)),
        scratch_shapes=[pltpu.VMEM((N, K), jnp.bfloat16)],
        compiler_params=pltpu.CompilerParams(
            dimension_semantics=("parallel", "arbitrary"),
            vmem_limit_bytes=62 * 1024 * 1024),
        cost_estimate=pl.CostEstimate(
            flops=2 * M * N * K,
            bytes_accessed=4 * M * K + 4 * K * N + 4 * M * N,
            transcendentals=0),
    )(w, x, b_row)
    return out


def kernel(x, w, b):
    bm = _BM if x.shape[0] % (2 * _BM) == 0 else 8
    return _forward(x, w, b, bm=bm)
